# baseline (speedup 1.0000x reference)
import jax
import jax.numpy as jnp
from jax import lax
from jax.experimental import pallas as pl
from jax.experimental.pallas import tpu as pltpu
from jax.experimental.pallas import tpu_sc as plsc

K = 1000
KP = 1024
NT = 20000
NS = 5000
D = 784
RB = 1000
NBLK = NT // RB
NC = 2
NSUB = 16
NW = NC * NSUB
CH = 160
GC = 16
NCHUNK = CH // GC
EPS = 1e-5
BIG = 2**30


def _tc_body(gt_ref, x_ref, t_ref, minv_ref, rep_ref, runm_ref, runi_ref):
    i = pl.program_id(0)
    x = x_ref[...]
    t = t_ref[...]
    inter = jnp.sum(x * t, axis=1, keepdims=True)
    union = (jnp.sum(x * x, axis=1, keepdims=True)
             + jnp.sum(t * t, axis=1, keepdims=True) + EPS)
    loss = 1.0 - 2.0 * inter / union
    gt = gt_ref[0, 0, :]
    kiota = lax.broadcasted_iota(jnp.int32, (RB, KP), 1)
    masked = jnp.where(gt[:, None] == kiota, loss, jnp.inf)
    bmin = jnp.min(masked, axis=0, keepdims=True)
    riota = lax.broadcasted_iota(jnp.int32, (RB, KP), 0)
    bidx = jnp.min(jnp.where(masked == bmin, riota, BIG),
                   axis=0, keepdims=True)

    @pl.when(i == 0)
    def _():
        runm_ref[...] = jnp.full((1, KP), jnp.inf, jnp.float32)
        runi_ref[...] = jnp.zeros((1, KP), jnp.int32)

    upd = bmin < runm_ref[...]
    runm_ref[...] = jnp.where(upd, bmin, runm_ref[...])
    runi_ref[...] = jnp.where(upd, bidx + i * RB, runi_ref[...])

    @pl.when(i == NBLK - 1)
    def _():
        minv_ref[...] = runm_ref[...]
        rep_ref[...] = runi_ref[...]


RB3 = 250
NBLK3 = NT // RB3


def _tc_body3(gt_ref, x_ref, t_ref, minv_ref, rep_ref, runm_ref, runi_ref):
    i = pl.program_id(0)
    x = x_ref[...]
    t = t_ref[...]
    inter = jnp.sum(jnp.sum(x * t, axis=2), axis=1, keepdims=True)
    union = (jnp.sum(jnp.sum(x * x, axis=2), axis=1, keepdims=True)
             + jnp.sum(jnp.sum(t * t, axis=2), axis=1, keepdims=True) + EPS)
    loss = 1.0 - 2.0 * inter / union
    gt = gt_ref[0, 0, :]
    kiota = lax.broadcasted_iota(jnp.int32, (RB3, KP), 1)
    masked = jnp.where(gt[:, None] == kiota, loss, jnp.inf)
    bmin = jnp.min(masked, axis=0, keepdims=True)
    riota = lax.broadcasted_iota(jnp.int32, (RB3, KP), 0)
    bidx = jnp.min(jnp.where(masked == bmin, riota, BIG),
                   axis=0, keepdims=True)

    @pl.when(i == 0)
    def _():
        runm_ref[...] = jnp.full((1, KP), jnp.inf, jnp.float32)
        runi_ref[...] = jnp.zeros((1, KP), jnp.int32)

    upd = bmin < runm_ref[...]
    runm_ref[...] = jnp.where(upd, bmin, runm_ref[...])
    runi_ref[...] = jnp.where(upd, bidx + i * RB3, runi_ref[...])

    @pl.when(i == NBLK3 - 1)
    def _():
        minv_ref[...] = runm_ref[...]
        rep_ref[...] = runi_ref[...]


def _tc_argmin3(gt3, pT3, gT3):
    return pl.pallas_call(
        _tc_body3,
        grid=(NBLK3,),
        in_specs=[
            pl.BlockSpec((1, 1, RB3), lambda i: (i, 0, 0)),
            pl.BlockSpec((RB3, 28, 28), lambda i: (i, 0, 0)),
            pl.BlockSpec((RB3, 28, 28), lambda i: (i, 0, 0)),
        ],
        out_specs=[
            pl.BlockSpec((1, KP), lambda i: (0, 0)),
            pl.BlockSpec((1, KP), lambda i: (0, 0)),
        ],
        out_shape=[
            jax.ShapeDtypeStruct((1, KP), jnp.float32),
            jax.ShapeDtypeStruct((1, KP), jnp.int32),
        ],
        scratch_shapes=[
            pltpu.VMEM((1, KP), jnp.float32),
            pltpu.VMEM((1, KP), jnp.int32),
        ],
    )(gt3, pT3, gT3)


def _tc_argmin(gt3, pT, gT):
    return pl.pallas_call(
        _tc_body,
        grid=(NBLK,),
        in_specs=[
            pl.BlockSpec((1, 1, RB), lambda i: (i, 0, 0)),
            pl.BlockSpec((RB, D), lambda i: (i, 0)),
            pl.BlockSpec((RB, D), lambda i: (i, 0)),
        ],
        out_specs=[
            pl.BlockSpec((1, KP), lambda i: (0, 0)),
            pl.BlockSpec((1, KP), lambda i: (0, 0)),
        ],
        out_shape=[
            jax.ShapeDtypeStruct((1, KP), jnp.float32),
            jax.ShapeDtypeStruct((1, KP), jnp.int32),
        ],
        scratch_shapes=[
            pltpu.VMEM((1, KP), jnp.float32),
            pltpu.VMEM((1, KP), jnp.int32),
        ],
    )(gt3, pT, gT)


def _vsum(v):
    parts = [v[i] for i in range(16)]
    while len(parts) > 1:
        parts = [parts[i] + parts[i + 1] for i in range(0, len(parts), 2)]
    return parts[0]


def _sc_body(minv_hbm, rep_hbm, gts_hbm, pT_hbm, pS_hbm, out_hbm,
             minv_v, rep_v, gs_v, ovec, tbuf, sbuf, sem_t, sem_s):
    wid = lax.axis_index("s") * NC + lax.axis_index("c")
    base = wid * CH
    pltpu.sync_copy(minv_hbm, minv_v.at[pl.ds(0, KP)])
    pltpu.sync_copy(rep_hbm, rep_v.at[pl.ds(0, KP)])
    pltpu.sync_copy(gts_hbm.at[pl.ds(base, CH)], gs_v)
    lanes = lax.iota(jnp.int32, 16)

    def chunk(c, total):
        gvec = gs_v[pl.ds(c * GC, GC)]
        rvec = jnp.zeros((GC,), jnp.int32)
        mvec = jnp.zeros((GC,), jnp.float32)
        for l in range(GC):
            g = gvec[l]
            rvec = jnp.where(lanes == l, rep_v[pl.ds(g, GC)][0], rvec)
            mvec = jnp.where(lanes == l, minv_v[pl.ds(g, GC)][0], mvec)
        jpos = base + c * GC + lanes
        valf = jnp.where((mvec < jnp.inf) & (jpos < NS),
                         jnp.float32(1.0), jnp.float32(0.0))
        srow = jnp.minimum(base + c * GC, NS - GC)
        cp_t = pltpu.async_copy(pT_hbm.at[rvec], tbuf, sem_t)
        cp_s = pltpu.async_copy(pS_hbm.at[pl.ds(srow, GC)], sbuf, sem_s)
        cp_t.wait()
        cp_s.wait()
        ivec = jnp.zeros((GC,), jnp.float32)
        uvec = jnp.ones((GC,), jnp.float32)
        for r in range(GC):
            def subrow(i, accs):
                aI, aX, aT = accs
                xa = sbuf[r, i, pl.ds(0, 16)]
                ta = tbuf[r, i, pl.ds(0, 16)]
                xb = sbuf[r, i, pl.ds(12, 16)]
                tb = tbuf[r, i, pl.ds(12, 16)]
                xb = jnp.where(lanes < 4, jnp.float32(0.0), xb)
                tb = jnp.where(lanes < 4, jnp.float32(0.0), tb)
                return (aI + xa * ta + xb * tb,
                        aX + xa * xa + xb * xb,
                        aT + ta * ta + tb * tb)
            z = jnp.zeros((16,), jnp.float32)
            aI, aX, aT = lax.fori_loop(0, 28, subrow, (z, z, z))
            ivec = jnp.where(lanes == r, _vsum(aI), ivec)
            uvec = jnp.where(lanes == r, _vsum(aX + aT) + EPS, uvec)
        pervec = 1.0 - 2.0 * ivec / uvec
        return total + valf * pervec

    total = lax.fori_loop(0, NCHUNK, chunk, jnp.zeros((16,), jnp.float32))
    ovec[...] = total
    pltpu.sync_copy(ovec, out_hbm.at[wid])


def _sc_call(minv, rep, gts_pad, pT, pS):
    mesh = plsc.VectorSubcoreMesh(core_axis_name="c", subcore_axis_name="s",
                                  num_cores=NC, num_subcores=NSUB)
    return pl.kernel(
        _sc_body,
        out_type=jax.ShapeDtypeStruct((NW, 16), jnp.float32),
        mesh=mesh,
        scratch_types=[
            pltpu.VMEM((KP + GC,), jnp.float32),
            pltpu.VMEM((KP + GC,), jnp.int32),
            pltpu.VMEM((CH,), jnp.int32),
            pltpu.VMEM((16,), jnp.float32),
            pltpu.VMEM((GC, 28, 28), jnp.float32),
            pltpu.VMEM((GC, 28, 28), jnp.float32),
            pltpu.SemaphoreType.DMA,
            pltpu.SemaphoreType.DMA,
        ],
        compiler_params=pltpu.CompilerParams(use_tc_tiling_on_sc=False),
    )(minv, rep, gts_pad, pT, pS)


def kernel(preds_T, preds_S, im_ind, gt_T, gt_S, iter, gt_inds_T, gt_inds_S):
    pT = preds_T.reshape(NT, D)
    gT = gt_T.reshape(NT, D)
    gt3 = gt_inds_T.reshape(NBLK, 1, RB)
    minv, rep = _tc_argmin(gt3, pT, gT)
    gts_pad = jnp.concatenate(
        [gt_inds_S, jnp.zeros((NW * CH - NS,), gt_inds_S.dtype)])
    part = _sc_call(minv.reshape(KP), rep.reshape(KP), gts_pad,
                    preds_T, preds_S)
    return jnp.sum(part)


# speedup vs baseline: 1.7118x; 1.7118x over previous
import jax
import jax.numpy as jnp
from jax import lax
from jax.experimental import pallas as pl
from jax.experimental.pallas import tpu as pltpu
from jax.experimental.pallas import tpu_sc as plsc

K = 1000
KP = 1024
NT = 20000
NS = 5000
D = 784
D2 = 896
RB = 1000
NBLK = NT // RB
NC = 2
NSUB = 16
NW = NC * NSUB
CH = 160
GC = 16
NCHUNK = CH // GC
EPS = 1e-5
BIG = 2**30


def _tc_body(gt_ref, x_ref, t_ref, minv_ref, rep_ref, runm_ref, runi_ref):
    i = pl.program_id(0)
    x = x_ref[...][:, :D]
    t = t_ref[...]
    inter = jnp.sum(x * t, axis=1, keepdims=True)
    union = (jnp.sum(x * x, axis=1, keepdims=True)
             + jnp.sum(t * t, axis=1, keepdims=True) + EPS)
    loss = 1.0 - 2.0 * inter / union
    gt = gt_ref[0, 0, :]
    kiota = lax.broadcasted_iota(jnp.int32, (RB, KP), 1)
    masked = jnp.where(gt[:, None] == kiota, loss, jnp.inf)
    bmin = jnp.min(masked, axis=0, keepdims=True)
    riota = lax.broadcasted_iota(jnp.int32, (RB, KP), 0)
    bidx = jnp.min(jnp.where(masked == bmin, riota, BIG),
                   axis=0, keepdims=True)

    @pl.when(i == 0)
    def _():
        runm_ref[...] = jnp.full((1, KP), jnp.inf, jnp.float32)
        runi_ref[...] = jnp.zeros((1, KP), jnp.int32)

    upd = bmin < runm_ref[...]
    runm_ref[...] = jnp.where(upd, bmin, runm_ref[...])
    runi_ref[...] = jnp.where(upd, bidx + i * RB, runi_ref[...])

    @pl.when(i == NBLK - 1)
    def _():
        minv_ref[...] = runm_ref[...]
        rep_ref[...] = runi_ref[...]


RB3 = 250
NBLK3 = NT // RB3


def _tc_body3(gt_ref, x_ref, t_ref, minv_ref, rep_ref, runm_ref, runi_ref):
    i = pl.program_id(0)
    x = x_ref[...]
    t = t_ref[...]
    inter = jnp.sum(jnp.sum(x * t, axis=2), axis=1, keepdims=True)
    union = (jnp.sum(jnp.sum(x * x, axis=2), axis=1, keepdims=True)
             + jnp.sum(jnp.sum(t * t, axis=2), axis=1, keepdims=True) + EPS)
    loss = 1.0 - 2.0 * inter / union
    gt = gt_ref[0, 0, :]
    kiota = lax.broadcasted_iota(jnp.int32, (RB3, KP), 1)
    masked = jnp.where(gt[:, None] == kiota, loss, jnp.inf)
    bmin = jnp.min(masked, axis=0, keepdims=True)
    riota = lax.broadcasted_iota(jnp.int32, (RB3, KP), 0)
    bidx = jnp.min(jnp.where(masked == bmin, riota, BIG),
                   axis=0, keepdims=True)

    @pl.when(i == 0)
    def _():
        runm_ref[...] = jnp.full((1, KP), jnp.inf, jnp.float32)
        runi_ref[...] = jnp.zeros((1, KP), jnp.int32)

    upd = bmin < runm_ref[...]
    runm_ref[...] = jnp.where(upd, bmin, runm_ref[...])
    runi_ref[...] = jnp.where(upd, bidx + i * RB3, runi_ref[...])

    @pl.when(i == NBLK3 - 1)
    def _():
        minv_ref[...] = runm_ref[...]
        rep_ref[...] = runi_ref[...]


def _tc_argmin3(gt3, pT3, gT3):
    return pl.pallas_call(
        _tc_body3,
        grid=(NBLK3,),
        in_specs=[
            pl.BlockSpec((1, 1, RB3), lambda i: (i, 0, 0)),
            pl.BlockSpec((RB3, 28, 28), lambda i: (i, 0, 0)),
            pl.BlockSpec((RB3, 28, 28), lambda i: (i, 0, 0)),
        ],
        out_specs=[
            pl.BlockSpec((1, KP), lambda i: (0, 0)),
            pl.BlockSpec((1, KP), lambda i: (0, 0)),
        ],
        out_shape=[
            jax.ShapeDtypeStruct((1, KP), jnp.float32),
            jax.ShapeDtypeStruct((1, KP), jnp.int32),
        ],
        scratch_shapes=[
            pltpu.VMEM((1, KP), jnp.float32),
            pltpu.VMEM((1, KP), jnp.int32),
        ],
    )(gt3, pT3, gT3)


def _tc_argmin(gt3, pT, gT):
    return pl.pallas_call(
        _tc_body,
        grid=(NBLK,),
        in_specs=[
            pl.BlockSpec((1, 1, RB), lambda i: (i, 0, 0)),
            pl.BlockSpec((RB, D2), lambda i: (i, 0)),
            pl.BlockSpec((RB, D), lambda i: (i, 0)),
        ],
        out_specs=[
            pl.BlockSpec((1, KP), lambda i: (0, 0)),
            pl.BlockSpec((1, KP), lambda i: (0, 0)),
        ],
        out_shape=[
            jax.ShapeDtypeStruct((1, KP), jnp.float32),
            jax.ShapeDtypeStruct((1, KP), jnp.int32),
        ],
        scratch_shapes=[
            pltpu.VMEM((1, KP), jnp.float32),
            pltpu.VMEM((1, KP), jnp.int32),
        ],
    )(gt3, pT, gT)


def _vsum(v):
    parts = [v[i] for i in range(16)]
    while len(parts) > 1:
        parts = [parts[i] + parts[i + 1] for i in range(0, len(parts), 2)]
    return parts[0]


def _sc_body(minv_hbm, rep_hbm, gts_hbm, pT_hbm, pS_hbm, out_hbm,
             minv_v, rep_v, gs_v, ovec, tbuf, sbuf, sem_t, sem_s):
    wid = lax.axis_index("s") * NC + lax.axis_index("c")
    base = wid * CH
    pltpu.sync_copy(minv_hbm, minv_v.at[pl.ds(0, KP)])
    pltpu.sync_copy(rep_hbm, rep_v.at[pl.ds(0, KP)])
    pltpu.sync_copy(gts_hbm.at[pl.ds(base, CH)], gs_v)
    lanes = lax.iota(jnp.int32, 16)

    def chunk(c, total):
        gvec = gs_v[pl.ds(c * GC, GC)]
        rvec = jnp.zeros((GC,), jnp.int32)
        mvec = jnp.zeros((GC,), jnp.float32)
        for l in range(GC):
            g = gvec[l]
            rvec = jnp.where(lanes == l, rep_v[pl.ds(g, GC)][0], rvec)
            mvec = jnp.where(lanes == l, minv_v[pl.ds(g, GC)][0], mvec)
        jpos = base + c * GC + lanes
        valf = jnp.where((mvec < jnp.inf) & (jpos < NS),
                         jnp.float32(1.0), jnp.float32(0.0))
        srow = jnp.minimum(base + c * GC, NS - GC)
        cp_t = pltpu.async_copy(pT_hbm.at[rvec], tbuf, sem_t)
        cp_s = pltpu.async_copy(pS_hbm.at[pl.ds(srow, GC)], sbuf, sem_s)
        cp_t.wait()
        cp_s.wait()
        ivec = jnp.zeros((GC,), jnp.float32)
        uvec = jnp.ones((GC,), jnp.float32)
        for r in range(GC):
            def col(k, accs):
                aI, aX, aT = accs
                xv = sbuf[r, pl.ds(k * 16, 16)]
                tv = tbuf[r, pl.ds(k * 16, 16)]
                return (aI + xv * tv, aX + xv * xv, aT + tv * tv)
            z = jnp.zeros((16,), jnp.float32)
            aI, aX, aT = lax.fori_loop(0, D2 // 16, col, (z, z, z))
            ivec = jnp.where(lanes == r, _vsum(aI), ivec)
            uvec = jnp.where(lanes == r, _vsum(aX + aT) + EPS, uvec)
        pervec = 1.0 - 2.0 * ivec / uvec
        return total + valf * pervec

    total = lax.fori_loop(0, NCHUNK, chunk, jnp.zeros((16,), jnp.float32))
    ovec[...] = total
    pltpu.sync_copy(ovec, out_hbm.at[wid])


def _sc_call(minv, rep, gts_pad, pT, pS):
    mesh = plsc.VectorSubcoreMesh(core_axis_name="c", subcore_axis_name="s",
                                  num_cores=NC, num_subcores=NSUB)
    return pl.kernel(
        _sc_body,
        out_type=jax.ShapeDtypeStruct((NW, 16), jnp.float32),
        mesh=mesh,
        scratch_types=[
            pltpu.VMEM((KP + GC,), jnp.float32),
            pltpu.VMEM((KP + GC,), jnp.int32),
            pltpu.VMEM((CH,), jnp.int32),
            pltpu.VMEM((16,), jnp.float32),
            pltpu.VMEM((GC, D2), jnp.float32),
            pltpu.VMEM((GC, D2), jnp.float32),
            pltpu.SemaphoreType.DMA,
            pltpu.SemaphoreType.DMA,
        ],
        compiler_params=pltpu.CompilerParams(use_tc_tiling_on_sc=True),
    )(minv, rep, gts_pad, pT, pS)


def kernel(preds_T, preds_S, im_ind, gt_T, gt_S, iter, gt_inds_T, gt_inds_S):
    pT = jnp.pad(preds_T.reshape(NT, D), ((0, 0), (0, D2 - D)))
    gT = gt_T.reshape(NT, D)
    pS = jnp.pad(preds_S.reshape(NS, D), ((0, 0), (0, D2 - D)))
    gt3 = gt_inds_T.reshape(NBLK, 1, RB)
    minv, rep = _tc_argmin(gt3, pT, gT)
    gts_pad = jnp.concatenate(
        [gt_inds_S, jnp.zeros((NW * CH - NS,), gt_inds_S.dtype)])
    part = _sc_call(minv.reshape(KP), rep.reshape(KP), gts_pad, pT, pS)
    return jnp.sum(part)


# speedup vs baseline: 2.9786x; 1.7400x over previous
import jax
import jax.numpy as jnp
from jax import lax
from jax.experimental import pallas as pl
from jax.experimental.pallas import tpu as pltpu
from jax.experimental.pallas import tpu_sc as plsc

K = 1000
KP = 1024
NT = 20000
NS = 5000
D2 = 896
NB = 1280
NBLK = 16
NTP = NB * NBLK
NBS = 1280
NBLKS = 4
NSP = NBS * NBLKS
NC = 2
NSUB = 16
NW = NC * NSUB
CH = 160
GC = 16
NCHUNK = CH // GC
EPS = 1e-5
BIG = 2**30


def _rowform(x, out_ref):
    nb = x.shape[2]
    zpad = jnp.zeros((nb, 4), jnp.float32)
    for i in range(28):
        piece = jnp.concatenate([x[i].T, zpad], axis=1)
        out_ref[:, pl.ds(32 * i, 32)] = piece


def _tc_body(gt_ref, x_ref, t_ref, minv_ref, rep_ref, rows_ref,
             runm_ref, runi_ref):
    i = pl.program_id(0)
    x = x_ref[...]
    t = t_ref[...]
    inter = jnp.sum(jnp.sum(x * t, axis=0), axis=0)[None, :]
    union = (jnp.sum(jnp.sum(x * x, axis=0), axis=0)
             + jnp.sum(jnp.sum(t * t, axis=0), axis=0))[None, :] + EPS
    loss = 1.0 - 2.0 * inter / union
    g = gt_ref[0, 0, :][None, :]
    kiota = lax.broadcasted_iota(jnp.int32, (KP, NB), 0)
    niota = lax.broadcasted_iota(jnp.int32, (KP, NB), 1)
    inb = (niota + i * NB) < NT
    masked = jnp.where((g == kiota) & inb, loss, jnp.inf)
    bmin = jnp.min(masked, axis=1, keepdims=True)
    bidx = jnp.min(jnp.where(masked == bmin, niota, BIG),
                   axis=1, keepdims=True)

    @pl.when(i == 0)
    def _():
        runm_ref[...] = jnp.full((KP, 1), jnp.inf, jnp.float32)
        runi_ref[...] = jnp.zeros((KP, 1), jnp.int32)

    upd = bmin < runm_ref[...]
    runm_ref[...] = jnp.where(upd, bmin, runm_ref[...])
    runi_ref[...] = jnp.where(upd, bidx + i * NB, runi_ref[...])

    @pl.when(i == NBLK - 1)
    def _():
        minv_ref[...] = runm_ref[...]
        rep_ref[...] = runi_ref[...]

    _rowform(x, rows_ref)


def _tc_argmin(gt3, pTt, gTt):
    return pl.pallas_call(
        _tc_body,
        grid=(NBLK,),
        in_specs=[
            pl.BlockSpec((1, 1, NB), lambda i: (0, 0, i)),
            pl.BlockSpec((28, 28, NB), lambda i: (0, 0, i)),
            pl.BlockSpec((28, 28, NB), lambda i: (0, 0, i)),
        ],
        out_specs=[
            pl.BlockSpec((KP, 1), lambda i: (0, 0)),
            pl.BlockSpec((KP, 1), lambda i: (0, 0)),
            pl.BlockSpec((NB, D2), lambda i: (i, 0)),
        ],
        out_shape=[
            jax.ShapeDtypeStruct((KP, 1), jnp.float32),
            jax.ShapeDtypeStruct((KP, 1), jnp.int32),
            jax.ShapeDtypeStruct((NTP, D2), jnp.float32),
        ],
        scratch_shapes=[
            pltpu.VMEM((KP, 1), jnp.float32),
            pltpu.VMEM((KP, 1), jnp.int32),
        ],
    )(gt3, pTt, gTt)


def _tc_rowform_body(x_ref, rows_ref):
    _rowform(x_ref[...], rows_ref)


def _tc_rowform_S(pSt):
    return pl.pallas_call(
        _tc_rowform_body,
        grid=(NBLKS,),
        in_specs=[pl.BlockSpec((28, 28, NBS), lambda i: (0, 0, i))],
        out_specs=pl.BlockSpec((NBS, D2), lambda i: (i, 0)),
        out_shape=jax.ShapeDtypeStruct((NSP, D2), jnp.float32),
    )(pSt)


def _vsum(v):
    parts = [v[i] for i in range(16)]
    while len(parts) > 1:
        parts = [parts[i] + parts[i + 1] for i in range(0, len(parts), 2)]
    return parts[0]


def _sc_body(minv_hbm, rep_hbm, gts_hbm, pT_hbm, pS_hbm, out_hbm,
             minv_v, rep_v, gs_v, ovec, tbuf, sbuf, sem_t, sem_s):
    wid = lax.axis_index("s") * NC + lax.axis_index("c")
    base = wid * CH
    pltpu.sync_copy(minv_hbm, minv_v.at[pl.ds(0, KP)])
    pltpu.sync_copy(rep_hbm, rep_v.at[pl.ds(0, KP)])
    pltpu.sync_copy(gts_hbm.at[pl.ds(base, CH)], gs_v)
    lanes = lax.iota(jnp.int32, 16)

    def chunk(c, total):
        gvec = gs_v[pl.ds(c * GC, GC)]
        rvec = jnp.zeros((GC,), jnp.int32)
        mvec = jnp.zeros((GC,), jnp.float32)
        for l in range(GC):
            g = gvec[l]
            rvec = jnp.where(lanes == l, rep_v[pl.ds(g, GC)][0], rvec)
            mvec = jnp.where(lanes == l, minv_v[pl.ds(g, GC)][0], mvec)
        jpos = base + c * GC + lanes
        valf = jnp.where((mvec < jnp.inf) & (jpos < NS),
                         jnp.float32(1.0), jnp.float32(0.0))
        srow = jnp.minimum(base + c * GC, NS - GC)
        cp_t = pltpu.async_copy(pT_hbm.at[rvec], tbuf, sem_t)
        cp_s = pltpu.async_copy(pS_hbm.at[pl.ds(srow, GC)], sbuf, sem_s)
        cp_t.wait()
        cp_s.wait()
        ivec = jnp.zeros((GC,), jnp.float32)
        uvec = jnp.ones((GC,), jnp.float32)
        for r in range(GC):
            def col(k, accs):
                aI, aX, aT = accs
                xv = sbuf[r, pl.ds(k * 16, 16)]
                tv = tbuf[r, pl.ds(k * 16, 16)]
                return (aI + xv * tv, aX + xv * xv, aT + tv * tv)
            z = jnp.zeros((16,), jnp.float32)
            aI, aX, aT = lax.fori_loop(0, D2 // 16, col, (z, z, z))
            ivec = jnp.where(lanes == r, _vsum(aI), ivec)
            uvec = jnp.where(lanes == r, _vsum(aX + aT) + EPS, uvec)
        pervec = 1.0 - 2.0 * ivec / uvec
        return total + valf * pervec

    total = lax.fori_loop(0, NCHUNK, chunk, jnp.zeros((16,), jnp.float32))
    ovec[...] = total
    pltpu.sync_copy(ovec, out_hbm.at[wid])


def _sc_call(minv, rep, gts_pad, pT, pS):
    mesh = plsc.VectorSubcoreMesh(core_axis_name="c", subcore_axis_name="s",
                                  num_cores=NC, num_subcores=NSUB)
    return pl.kernel(
        _sc_body,
        out_type=jax.ShapeDtypeStruct((NW, 16), jnp.float32),
        mesh=mesh,
        scratch_types=[
            pltpu.VMEM((KP + GC,), jnp.float32),
            pltpu.VMEM((KP + GC,), jnp.int32),
            pltpu.VMEM((CH,), jnp.int32),
            pltpu.VMEM((16,), jnp.float32),
            pltpu.VMEM((GC, D2), jnp.float32),
            pltpu.VMEM((GC, D2), jnp.float32),
            pltpu.SemaphoreType.DMA,
            pltpu.SemaphoreType.DMA,
        ],
        compiler_params=pltpu.CompilerParams(use_tc_tiling_on_sc=True),
    )(minv, rep, gts_pad, pT, pS)


def kernel(preds_T, preds_S, im_ind, gt_T, gt_S, iter, gt_inds_T, gt_inds_S):
    pTt = jnp.transpose(preds_T, (1, 2, 0))
    gTt = jnp.transpose(gt_T, (1, 2, 0))
    pSt = jnp.transpose(preds_S, (1, 2, 0))
    gt3 = gt_inds_T.reshape(1, 1, NT)
    minv, rep, pT_rows = _tc_argmin(gt3, pTt, gTt)
    pS_rows = _tc_rowform_S(pSt)
    gts_pad = jnp.concatenate(
        [gt_inds_S, jnp.zeros((NW * CH - NS,), gt_inds_S.dtype)])
    part = _sc_call(minv.reshape(KP), rep.reshape(KP), gts_pad,
                    pT_rows, pS_rows)
    return jnp.sum(part)


# speedup vs baseline: 5.1189x; 1.7186x over previous
import jax
import jax.numpy as jnp
from jax import lax
from jax.experimental import pallas as pl
from jax.experimental.pallas import tpu as pltpu
from jax.experimental.pallas import tpu_sc as plsc

K = 1000
KP = 1024
NT = 20000
NS = 5000
D2 = 896
NB = 1280
NBLK = 16
NTP = NB * NBLK
NBS = 1280
NBLKS = 4
NSP = NBS * NBLKS
NC = 2
NSUB = 16
NW = NC * NSUB
CH = 160
GC = 16
NCHUNK = CH // GC
EPS = 1e-5
BIG = 2**30


def _rowform(x, out_ref):
    nb = x.shape[2]
    xp = jnp.concatenate([x, jnp.zeros((28, 4, nb), jnp.float32)], axis=1)
    for i in range(7):
        quad = xp[4 * i:4 * i + 4].reshape(128, nb)
        out_ref[:, pl.ds(128 * i, 128)] = quad.T


def _tc_body(gt_ref, x_ref, t_ref, minv_ref, rep_ref, rows_ref,
             runm_ref, runi_ref):
    i = pl.program_id(0)
    x = x_ref[...]
    t = t_ref[...]
    inter = jnp.sum(jnp.sum(x * t, axis=0), axis=0)[None, :]
    union = (jnp.sum(jnp.sum(x * x, axis=0), axis=0)
             + jnp.sum(jnp.sum(t * t, axis=0), axis=0))[None, :] + EPS
    loss = 1.0 - 2.0 * inter / union
    g = gt_ref[0, 0, :][None, :]
    kiota = lax.broadcasted_iota(jnp.int32, (KP, NB), 0)
    niota = lax.broadcasted_iota(jnp.int32, (KP, NB), 1)
    inb = (niota + i * NB) < NT
    masked = jnp.where((g == kiota) & inb, loss, jnp.inf)
    bmin = jnp.min(masked, axis=1, keepdims=True)
    bidx = jnp.min(jnp.where(masked == bmin, niota, BIG),
                   axis=1, keepdims=True)

    @pl.when(i == 0)
    def _():
        runm_ref[...] = jnp.full((KP, 1), jnp.inf, jnp.float32)
        runi_ref[...] = jnp.zeros((KP, 1), jnp.int32)

    upd = bmin < runm_ref[...]
    runm_ref[...] = jnp.where(upd, bmin, runm_ref[...])
    runi_ref[...] = jnp.where(upd, bidx + i * NB, runi_ref[...])

    @pl.when(i == NBLK - 1)
    def _():
        minv_ref[...] = runm_ref[...]
        rep_ref[...] = runi_ref[...]

    _rowform(x, rows_ref)


def _tc_argmin(gt3, pTt, gTt):
    return pl.pallas_call(
        _tc_body,
        grid=(NBLK,),
        in_specs=[
            pl.BlockSpec((1, 1, NB), lambda i: (0, 0, i)),
            pl.BlockSpec((28, 28, NB), lambda i: (0, 0, i)),
            pl.BlockSpec((28, 28, NB), lambda i: (0, 0, i)),
        ],
        out_specs=[
            pl.BlockSpec((KP, 1), lambda i: (0, 0)),
            pl.BlockSpec((KP, 1), lambda i: (0, 0)),
            pl.BlockSpec((NB, D2), lambda i: (i, 0)),
        ],
        out_shape=[
            jax.ShapeDtypeStruct((KP, 1), jnp.float32),
            jax.ShapeDtypeStruct((KP, 1), jnp.int32),
            jax.ShapeDtypeStruct((NTP, D2), jnp.float32),
        ],
        scratch_shapes=[
            pltpu.VMEM((KP, 1), jnp.float32),
            pltpu.VMEM((KP, 1), jnp.int32),
        ],
    )(gt3, pTt, gTt)


def _tc_rowform_body(x_ref, rows_ref):
    _rowform(x_ref[...], rows_ref)


def _tc_rowform_S(pSt):
    return pl.pallas_call(
        _tc_rowform_body,
        grid=(NBLKS,),
        in_specs=[pl.BlockSpec((28, 28, NBS), lambda i: (0, 0, i))],
        out_specs=pl.BlockSpec((NBS, D2), lambda i: (i, 0)),
        out_shape=jax.ShapeDtypeStruct((NSP, D2), jnp.float32),
    )(pSt)


def _vsum(v):
    parts = [v[i] for i in range(16)]
    while len(parts) > 1:
        parts = [parts[i] + parts[i + 1] for i in range(0, len(parts), 2)]
    return parts[0]


def _sc_body(minv_hbm, rep_hbm, gts_hbm, pT_hbm, pS_hbm, out_hbm,
             minv_v, rep_v, gs_v, ovec, tbuf, sbuf, sem_t, sem_s):
    wid = lax.axis_index("s") * NC + lax.axis_index("c")
    base = wid * CH
    pltpu.sync_copy(minv_hbm, minv_v.at[pl.ds(0, KP)])
    pltpu.sync_copy(rep_hbm, rep_v.at[pl.ds(0, KP)])
    pltpu.sync_copy(gts_hbm.at[pl.ds(base, CH)], gs_v)
    lanes = lax.iota(jnp.int32, 16)

    def chunk(c, total):
        gvec = gs_v[pl.ds(c * GC, GC)]
        rvec = jnp.zeros((GC,), jnp.int32)
        mvec = jnp.zeros((GC,), jnp.float32)
        for l in range(GC):
            g = gvec[l]
            rvec = jnp.where(lanes == l, rep_v[pl.ds(g, GC)][0], rvec)
            mvec = jnp.where(lanes == l, minv_v[pl.ds(g, GC)][0], mvec)
        jpos = base + c * GC + lanes
        valf = jnp.where((mvec < jnp.inf) & (jpos < NS),
                         jnp.float32(1.0), jnp.float32(0.0))
        srow = jnp.minimum(base + c * GC, NS - GC)
        cp_t = pltpu.async_copy(pT_hbm.at[rvec], tbuf, sem_t)
        cp_s = pltpu.async_copy(pS_hbm.at[pl.ds(srow, GC)], sbuf, sem_s)
        cp_t.wait()
        cp_s.wait()
        ivec = jnp.zeros((GC,), jnp.float32)
        uvec = jnp.ones((GC,), jnp.float32)
        for r in range(GC):
            def col(k, accs):
                aI, aX, aT = accs
                xv = sbuf[r, pl.ds(k * 16, 16)]
                tv = tbuf[r, pl.ds(k * 16, 16)]
                return (aI + xv * tv, aX + xv * xv, aT + tv * tv)
            z = jnp.zeros((16,), jnp.float32)
            aI, aX, aT = lax.fori_loop(0, D2 // 16, col, (z, z, z))
            ivec = jnp.where(lanes == r, _vsum(aI), ivec)
            uvec = jnp.where(lanes == r, _vsum(aX + aT) + EPS, uvec)
        pervec = 1.0 - 2.0 * ivec / uvec
        return total + valf * pervec

    total = lax.fori_loop(0, NCHUNK, chunk, jnp.zeros((16,), jnp.float32))
    ovec[...] = total
    pltpu.sync_copy(ovec, out_hbm.at[wid])


def _sc_call(minv, rep, gts_pad, pT, pS):
    mesh = plsc.VectorSubcoreMesh(core_axis_name="c", subcore_axis_name="s",
                                  num_cores=NC, num_subcores=NSUB)
    return pl.kernel(
        _sc_body,
        out_type=jax.ShapeDtypeStruct((NW, 16), jnp.float32),
        mesh=mesh,
        scratch_types=[
            pltpu.VMEM((KP + GC,), jnp.float32),
            pltpu.VMEM((KP + GC,), jnp.int32),
            pltpu.VMEM((CH,), jnp.int32),
            pltpu.VMEM((16,), jnp.float32),
            pltpu.VMEM((GC, D2), jnp.float32),
            pltpu.VMEM((GC, D2), jnp.float32),
            pltpu.SemaphoreType.DMA,
            pltpu.SemaphoreType.DMA,
        ],
        compiler_params=pltpu.CompilerParams(use_tc_tiling_on_sc=True),
    )(minv, rep, gts_pad, pT, pS)


def kernel(preds_T, preds_S, im_ind, gt_T, gt_S, iter, gt_inds_T, gt_inds_S):
    pTt = jnp.transpose(preds_T, (1, 2, 0))
    gTt = jnp.transpose(gt_T, (1, 2, 0))
    pSt = jnp.transpose(preds_S, (1, 2, 0))
    gt3 = gt_inds_T.reshape(1, 1, NT)
    minv, rep, pT_rows = _tc_argmin(gt3, pTt, gTt)
    pS_rows = _tc_rowform_S(pSt)
    gts_pad = jnp.concatenate(
        [gt_inds_S, jnp.zeros((NW * CH - NS,), gt_inds_S.dtype)])
    part = _sc_call(minv.reshape(KP), rep.reshape(KP), gts_pad,
                    pT_rows, pS_rows)
    return jnp.sum(part)


# speedup vs baseline: 5.4701x; 1.0686x over previous
import jax
import jax.numpy as jnp
from jax import lax
from jax.experimental import pallas as pl
from jax.experimental.pallas import tpu as pltpu
from jax.experimental.pallas import tpu_sc as plsc

K = 1000
KP = 1024
NT = 20000
NS = 5000
D2 = 896
NB = 1280
NBLK = 16
NTP = NB * NBLK
NBS = 1280
NBLKS = 4
NSP = NBS * NBLKS
NC = 2
NSUB = 16
NW = NC * NSUB
CH = 160
GC = 16
NCHUNK = CH // GC
EPS = 1e-5
BIG = 2**30


def _rowform(x, out_ref):
    nb = x.shape[2]
    xp = jnp.concatenate([x, jnp.zeros((28, 4, nb), jnp.float32)], axis=1)
    for i in range(7):
        quad = xp[4 * i:4 * i + 4].reshape(128, nb)
        out_ref[:, pl.ds(128 * i, 128)] = quad.T


def _tc_body(gt_ref, x_ref, t_ref, minv_ref, rep_ref, rows_ref,
             runm_ref, runi_ref):
    i = pl.program_id(0)
    x = x_ref[...]
    t = t_ref[...]
    inter = jnp.sum(jnp.sum(x * t, axis=0), axis=0)[None, :]
    union = (jnp.sum(jnp.sum(x * x, axis=0), axis=0)
             + jnp.sum(jnp.sum(t * t, axis=0), axis=0))[None, :] + EPS
    loss = 1.0 - 2.0 * inter / union
    g = gt_ref[0, 0, :][None, :]
    kiota = lax.broadcasted_iota(jnp.int32, (KP, NB), 0)
    niota = lax.broadcasted_iota(jnp.int32, (KP, NB), 1)
    inb = (niota + i * NB) < NT
    masked = jnp.where((g == kiota) & inb, loss, jnp.inf)
    bmin = jnp.min(masked, axis=1, keepdims=True)
    bidx = jnp.min(jnp.where(masked == bmin, niota, BIG),
                   axis=1, keepdims=True)

    @pl.when(i == 0)
    def _():
        runm_ref[...] = jnp.full((KP, 1), jnp.inf, jnp.float32)
        runi_ref[...] = jnp.zeros((KP, 1), jnp.int32)

    upd = bmin < runm_ref[...]
    runm_ref[...] = jnp.where(upd, bmin, runm_ref[...])
    runi_ref[...] = jnp.where(upd, bidx + i * NB, runi_ref[...])

    @pl.when(i == NBLK - 1)
    def _():
        minv_ref[...] = runm_ref[...]
        rep_ref[...] = runi_ref[...]

    _rowform(x, rows_ref)


def _tc_argmin(gt3, pTt, gTt):
    return pl.pallas_call(
        _tc_body,
        grid=(NBLK,),
        in_specs=[
            pl.BlockSpec((1, 1, NB), lambda i: (0, 0, i)),
            pl.BlockSpec((28, 28, NB), lambda i: (0, 0, i)),
            pl.BlockSpec((28, 28, NB), lambda i: (0, 0, i)),
        ],
        out_specs=[
            pl.BlockSpec((KP, 1), lambda i: (0, 0)),
            pl.BlockSpec((KP, 1), lambda i: (0, 0)),
            pl.BlockSpec((NB, D2), lambda i: (i, 0)),
        ],
        out_shape=[
            jax.ShapeDtypeStruct((KP, 1), jnp.float32),
            jax.ShapeDtypeStruct((KP, 1), jnp.int32),
            jax.ShapeDtypeStruct((NTP, D2), jnp.float32),
        ],
        scratch_shapes=[
            pltpu.VMEM((KP, 1), jnp.float32),
            pltpu.VMEM((KP, 1), jnp.int32),
        ],
    )(gt3, pTt, gTt)


def _tc_rowform_body(x_ref, rows_ref):
    _rowform(x_ref[...], rows_ref)


def _tc_rowform_S(pSt):
    return pl.pallas_call(
        _tc_rowform_body,
        grid=(NBLKS,),
        in_specs=[pl.BlockSpec((28, 28, NBS), lambda i: (0, 0, i))],
        out_specs=pl.BlockSpec((NBS, D2), lambda i: (i, 0)),
        out_shape=jax.ShapeDtypeStruct((NSP, D2), jnp.float32),
    )(pSt)


def _vsum(v):
    parts = [v[i] for i in range(16)]
    while len(parts) > 1:
        parts = [parts[i] + parts[i + 1] for i in range(0, len(parts), 2)]
    return parts[0]


def _sc_body(minv_hbm, rep_hbm, gts_hbm, pT_hbm, pS_hbm, out_hbm,
             minv_v, rep_v, gs_v, ovec, tbuf0, sbuf0, tbuf1, sbuf1,
             semt0, sems0, semt1, sems1):
    wid = lax.axis_index("s") * NC + lax.axis_index("c")
    base = wid * CH
    pltpu.sync_copy(minv_hbm, minv_v.at[pl.ds(0, KP)])
    pltpu.sync_copy(rep_hbm, rep_v.at[pl.ds(0, KP)])
    pltpu.sync_copy(gts_hbm.at[pl.ds(base, CH)], gs_v.at[pl.ds(0, CH)])
    lanes = lax.iota(jnp.int32, 16)

    def compose(c):
        gvec = gs_v[pl.ds(c * GC, GC)]
        gvec = jnp.clip(gvec, 0, KP - 1)
        rvec = jnp.zeros((GC,), jnp.int32)
        mvec = jnp.zeros((GC,), jnp.float32)
        for l in range(GC):
            g = gvec[l]
            rvec = jnp.where(lanes == l, rep_v[pl.ds(g, GC)][0], rvec)
            mvec = jnp.where(lanes == l, minv_v[pl.ds(g, GC)][0], mvec)
        jpos = base + c * GC + lanes
        valf = jnp.where((mvec < jnp.inf) & (jpos < NS),
                         jnp.float32(1.0), jnp.float32(0.0))
        srow = jnp.minimum(base + c * GC, NS - GC)
        return rvec, valf, srow

    def start(rvec, srow, tb, sb, st, ss):
        pltpu.async_copy(pT_hbm.at[rvec], tb, st)
        pltpu.async_copy(pS_hbm.at[pl.ds(srow, GC)], sb, ss)

    def drain(tb, sb, st, ss):
        pltpu.make_async_copy(pT_hbm.at[lanes], tb, st).wait()
        pltpu.make_async_copy(pS_hbm.at[pl.ds(0, GC)], sb, ss).wait()

    def compute(tb, sb, valf, total):
        ivec = jnp.zeros((GC,), jnp.float32)
        uvec = jnp.ones((GC,), jnp.float32)
        for r in range(GC):
            def col(k, accs):
                aI, aX, aT = accs
                for u in range(7):
                    off = k * 112 + u * 16
                    xv = sb[r, pl.ds(off, 16)]
                    tv = tb[r, pl.ds(off, 16)]
                    aI = aI + xv * tv
                    aX = aX + xv * xv
                    aT = aT + tv * tv
                return (aI, aX, aT)
            z = jnp.zeros((16,), jnp.float32)
            aI, aX, aT = lax.fori_loop(0, D2 // 112, col, (z, z, z))
            ivec = jnp.where(lanes == r, _vsum(aI), ivec)
            uvec = jnp.where(lanes == r, _vsum(aX + aT) + EPS, uvec)
        pervec = 1.0 - 2.0 * ivec / uvec
        return total + valf * pervec

    rv0, vf0, sr0 = compose(0)
    start(rv0, sr0, tbuf0, sbuf0, semt0, sems0)

    def pair(i, carry):
        total, vf_a = carry
        c0 = 2 * i
        rv1, vf1, sr1 = compose(c0 + 1)
        start(rv1, sr1, tbuf1, sbuf1, semt1, sems1)
        drain(tbuf0, sbuf0, semt0, sems0)
        total = compute(tbuf0, sbuf0, vf_a, total)
        rv2, vf2, sr2 = compose(c0 + 2)

        @pl.when(i < NCHUNK // 2 - 1)
        def _():
            start(rv2, sr2, tbuf0, sbuf0, semt0, sems0)

        drain(tbuf1, sbuf1, semt1, sems1)
        total = compute(tbuf1, sbuf1, vf1, total)
        return total, vf2

    total, _ = lax.fori_loop(0, NCHUNK // 2, pair,
                             (jnp.zeros((16,), jnp.float32), vf0))
    ovec[...] = total
    pltpu.sync_copy(ovec, out_hbm.at[wid])


def _sc_call(minv, rep, gts_pad, pT, pS):
    mesh = plsc.VectorSubcoreMesh(core_axis_name="c", subcore_axis_name="s",
                                  num_cores=NC, num_subcores=NSUB)
    return pl.kernel(
        _sc_body,
        out_type=jax.ShapeDtypeStruct((NW, 16), jnp.float32),
        mesh=mesh,
        scratch_types=[
            pltpu.VMEM((KP + GC,), jnp.float32),
            pltpu.VMEM((KP + GC,), jnp.int32),
            pltpu.VMEM((CH + GC,), jnp.int32),
            pltpu.VMEM((16,), jnp.float32),
            pltpu.VMEM((GC, D2), jnp.float32),
            pltpu.VMEM((GC, D2), jnp.float32),
            pltpu.VMEM((GC, D2), jnp.float32),
            pltpu.VMEM((GC, D2), jnp.float32),
            pltpu.SemaphoreType.DMA,
            pltpu.SemaphoreType.DMA,
            pltpu.SemaphoreType.DMA,
            pltpu.SemaphoreType.DMA,
        ],
        compiler_params=pltpu.CompilerParams(use_tc_tiling_on_sc=True),
    )(minv, rep, gts_pad, pT, pS)


def kernel(preds_T, preds_S, im_ind, gt_T, gt_S, iter, gt_inds_T, gt_inds_S):
    pTt = jnp.transpose(preds_T, (1, 2, 0))
    gTt = jnp.transpose(gt_T, (1, 2, 0))
    pSt = jnp.transpose(preds_S, (1, 2, 0))
    gt3 = gt_inds_T.reshape(1, 1, NT)
    minv, rep, pT_rows = _tc_argmin(gt3, pTt, gTt)
    pS_rows = _tc_rowform_S(pSt)
    gts_pad = jnp.concatenate(
        [gt_inds_S, jnp.zeros((NW * CH - NS,), gt_inds_S.dtype)])
    part = _sc_call(minv.reshape(KP), rep.reshape(KP), gts_pad,
                    pT_rows, pS_rows)
    return jnp.sum(part)
